# baseline (speedup 1.0000x reference)
import functools

import jax
import jax.numpy as jnp
from jax import lax
from jax.experimental import pallas as pl
from jax.experimental.pallas import tpu as pltpu
from jax.experimental.pallas import tpu_sc as plsc

VOCAB = 100000
EMBED_DIM = 128
HIDDEN = 128
NUM_CLASSES = 2
BATCH = 4096
SEQ = 200
SEQ_PAD = 208
HALF = SEQ_PAD // 2

NUM_CORES = 2
NUM_SUBCORES = 16
NUM_WORKERS = NUM_CORES * NUM_SUBCORES
ROWS_PER_WORKER = BATCH // NUM_WORKERS
ROWS_PER_CORE = BATCH // NUM_CORES
CHUNKS = ROWS_PER_WORKER * 2

NBUF = 4
INFLIGHT = 2
UNROLL = 4


def _sc_pool_body(xg_hbm, dst_hbm, table_hbm, out_hbm,
                  idx_v, dst_v, b0, b1, b2, b3, acc_sh, gsem, ssem):
    c = lax.axis_index("c")
    s = lax.axis_index("s")
    bufs = (b0, b1, b2, b3)
    wid = c * NUM_SUBCORES + s
    chunk0 = wid * CHUNKS

    pltpu.sync_copy(xg_hbm.at[pl.ds(chunk0, CHUNKS)], idx_v)
    pltpu.sync_copy(dst_hbm.at[pl.ds(wid * ROWS_PER_WORKER,
                                     ROWS_PER_WORKER)], dst_v)

    def zrow(i, carry):
        for ci in range(EMBED_DIM // 16):
            b0[i, pl.ds(ci * 16, 16)] = jnp.zeros((16,), jnp.float32)
        return carry
    lax.fori_loop(0, HALF, zrow, 0)
    arow = s * ROWS_PER_WORKER
    pltpu.sync_copy(b0, acc_sh.at[pl.ds(arow, HALF)])
    pltpu.sync_copy(b0.at[pl.ds(0, ROWS_PER_WORKER - HALF)],
                    acc_sh.at[pl.ds(arow + HALF, ROWS_PER_WORKER - HALF)])

    def gather(g, buf):
        return pltpu.async_copy(table_hbm.at[idx_v.at[g]], buf, gsem)

    def scat(g, buf):
        return pltpu.async_copy(buf, acc_sh.at[dst_v.at[g // 2]], ssem,
                                add=True)

    for b in range(INFLIGHT):
        gather(jnp.int32(b), bufs[b])

    def outer(go, carry):
        for b in range(UNROLL):
            g = go * UNROLL + b
            buf = bufs[b]
            pltpu.make_async_copy(table_hbm.at[idx_v.at[g]], buf, gsem).wait()

            @pl.when(g >= INFLIGHT)
            def _():
                pltpu.make_async_copy(
                    bufs[(b - INFLIGHT) % NBUF],
                    acc_sh.at[dst_v.at[(g - INFLIGHT) // 2]], ssem).wait()

            scat(g, buf)

            @pl.when(g + INFLIGHT < CHUNKS)
            def _():
                gather(g + INFLIGHT, bufs[(b + INFLIGHT) % NBUF])
        return carry

    lax.fori_loop(0, CHUNKS // UNROLL, outer, 0)

    for b in range(INFLIGHT):
        g = CHUNKS - INFLIGHT + b
        pltpu.make_async_copy(
            bufs[g % NBUF], acc_sh.at[dst_v.at[g // 2]], ssem).wait()

    pltpu.sync_copy(acc_sh.at[pl.ds(arow, ROWS_PER_WORKER)],
                    out_hbm.at[pl.ds(c * ROWS_PER_CORE + arow,
                                     ROWS_PER_WORKER)])


@jax.jit
def _sc_pool(xg, dst, table):
    mesh = plsc.VectorSubcoreMesh(core_axis_name="c", subcore_axis_name="s")
    f = functools.partial(
        pl.kernel, mesh=mesh,
        out_type=jax.ShapeDtypeStruct((BATCH, EMBED_DIM), jnp.float32),
        scratch_types=[
            pltpu.VMEM((CHUNKS, HALF), jnp.int32),
            pltpu.VMEM((ROWS_PER_WORKER, HALF), jnp.int32),
            pltpu.VMEM((HALF, EMBED_DIM), jnp.float32),
            pltpu.VMEM((HALF, EMBED_DIM), jnp.float32),
            pltpu.VMEM((HALF, EMBED_DIM), jnp.float32),
            pltpu.VMEM((HALF, EMBED_DIM), jnp.float32),
            pltpu.VMEM_SHARED((ROWS_PER_CORE, EMBED_DIM), jnp.float32),
            pltpu.SemaphoreType.DMA,
            pltpu.SemaphoreType.DMA,
        ],
    )(_sc_pool_body)
    return f(xg, dst, table)


def _mlp_body(acc_ref, mask_ref, row0_ref, w1t_ref, b1_ref, g_ref, bt_ref,
              w2t_ref, b2_ref, out_ref):
    acc = acc_ref[...]
    maskf = mask_ref[...].astype(jnp.float32)
    valid = SEQ - jnp.sum(maskf, axis=1, keepdims=True)
    sentinel_cnt = SEQ_PAD - valid
    lengths = jnp.maximum(valid, 1.0)
    sv = (acc - sentinel_cnt * row0_ref[...]) / lengths
    h = jnp.dot(sv, w1t_ref[...], preferred_element_type=jnp.float32)
    h = h + b1_ref[...]
    mu = jnp.mean(h, axis=-1, keepdims=True)
    var = jnp.mean(jnp.square(h), axis=-1, keepdims=True) - jnp.square(mu)
    hn = (h - mu) * jax.lax.rsqrt(var + 1e-5) * g_ref[...] + bt_ref[...]
    hr = jnp.maximum(hn, 0.0)
    out_ref[...] = jnp.dot(hr, w2t_ref[...],
                           preferred_element_type=jnp.float32) + b2_ref[...]


@jax.jit
def _tc_mlp(acc, mask, row0, w1t, b1, gamma, beta, w2t_pad, b2_pad):
    bm = 512
    grid = (BATCH // bm,)
    return pl.pallas_call(
        _mlp_body,
        grid=grid,
        in_specs=[
            pl.BlockSpec((bm, EMBED_DIM), lambda i: (i, 0)),
            pl.BlockSpec((bm, SEQ), lambda i: (i, 0)),
            pl.BlockSpec((1, EMBED_DIM), lambda i: (0, 0)),
            pl.BlockSpec((EMBED_DIM, HIDDEN), lambda i: (0, 0)),
            pl.BlockSpec((1, HIDDEN), lambda i: (0, 0)),
            pl.BlockSpec((1, HIDDEN), lambda i: (0, 0)),
            pl.BlockSpec((1, HIDDEN), lambda i: (0, 0)),
            pl.BlockSpec((HIDDEN, HIDDEN), lambda i: (0, 0)),
            pl.BlockSpec((1, HIDDEN), lambda i: (0, 0)),
        ],
        out_specs=pl.BlockSpec((bm, HIDDEN), lambda i: (i, 0)),
        out_shape=jax.ShapeDtypeStruct((BATCH, HIDDEN), jnp.float32),
    )(acc, mask, row0, w1t, b1, gamma, beta, w2t_pad, b2_pad)


def kernel(x, mask, table, W1, b1, gamma, beta, W2, b2):
    x = x.astype(jnp.int32)
    safe_x = jnp.where(mask, 0, x)
    xg = jnp.pad(safe_x, ((0, 0), (0, SEQ_PAD - SEQ))).reshape(
        BATCH * 2, HALF)
    dst = jnp.broadcast_to(
        (jnp.arange(BATCH, dtype=jnp.int32) % ROWS_PER_CORE)[:, None],
        (BATCH, HALF)) + jnp.zeros((BATCH, HALF), jnp.int32)
    acc = _sc_pool(xg, dst, table)

    row0 = table[0].reshape(1, EMBED_DIM)
    w1t = W1.T
    w2t_pad = jnp.zeros((HIDDEN, HIDDEN), jnp.float32).at[:, :NUM_CLASSES].set(W2.T)
    b2_pad = jnp.zeros((1, HIDDEN), jnp.float32).at[:, :NUM_CLASSES].set(b2)
    out = _tc_mlp(acc, mask, row0, w1t, b1.reshape(1, -1),
                  gamma.reshape(1, -1), beta.reshape(1, -1), w2t_pad, b2_pad)
    return out[:, :NUM_CLASSES]
